# Initial kernel scaffold; baseline (speedup 1.0000x reference)
import jax
import jax.numpy as jnp
from jax.experimental import pallas as pl


def kernel(user_seq, spd_table, emb):
    raise NotImplementedError("write your pallas kernel here")


# speedup vs baseline: 13.2597x; 13.2597x over previous
import functools

import jax
import jax.numpy as jnp
from jax import lax
from jax.experimental import pallas as pl
from jax.experimental.pallas import tpu as pltpu
from jax.experimental.pallas import tpu_sc as plsc

NUM_NODES = 4096
H = 16
B = 1024
L = 50
LL = L * L
G = (LL + 15) // 16
GP = 160
LLP = GP * 16
LP = 64
NW = 32
BPW = B // NW
VT = NUM_NODES + 1


def _sc_body(seq_hbm, spd_hbm, embt_hbm, out_hbm,
             seq_v, idx_v, spd_v, embt_v, out_v, sem):
    wid = lax.axis_index("s") * 2 + lax.axis_index("c")
    pltpu.sync_copy(embt_hbm, embt_v)
    iota = lax.iota(jnp.int32, 16)

    def batch_body(t, carry):
        b = wid * BPW + t
        pltpu.sync_copy(seq_hbm.at[b], seq_v)

        def idx_body(g, c):
            di, mo = c
            hi = plsc.load_gather(seq_v, [di])
            lo = plsc.load_gather(seq_v, [mo])
            idx_v[pl.ds(g * 16, 16)] = hi * NUM_NODES + lo
            mo2 = mo + 16
            over = mo2 >= L
            mo2 = jnp.where(over, mo2 - L, mo2)
            return (di + over.astype(jnp.int32), mo2)

        lax.fori_loop(0, GP, idx_body, (iota * 0, iota))

        pltpu.async_copy(spd_hbm.at[idx_v], spd_v, sem).wait()

        def g_body(g, c):
            sv = spd_v[pl.ds(g * 16, 16)]
            base = g * 16 + iota
            for h in range(H):
                val = plsc.load_gather(embt_v, [sv + (h * VT)])
                plsc.store_scatter(out_v, [base + (h * LL)], val)
            return c

        lax.fori_loop(0, G - 1, g_body, 0)

        g = G - 1
        sv = spd_v[pl.ds(g * 16, 16)]
        base = g * 16 + iota
        tmask = iota < (LL - g * 16)
        for h in range(H):
            val = plsc.load_gather(embt_v, [sv + (h * VT)])
            plsc.store_scatter(out_v, [base + (h * LL)], val, mask=tmask)

        pltpu.sync_copy(out_v, out_hbm.at[b])
        return carry

    lax.fori_loop(0, BPW, batch_body, 0)


@functools.partial(
    pl.kernel,
    mesh=plsc.VectorSubcoreMesh(core_axis_name="c", subcore_axis_name="s"),
    compiler_params=pltpu.CompilerParams(needs_layout_passes=False),
    out_type=jax.ShapeDtypeStruct((B, H * LL), jnp.float32),
    scratch_types=[
        pltpu.VMEM((LP,), jnp.int32),
        pltpu.VMEM((LLP,), jnp.int32),
        pltpu.VMEM((LLP,), jnp.int32),
        pltpu.VMEM((H * VT,), jnp.float32),
        pltpu.VMEM((H * LL,), jnp.float32),
        pltpu.SemaphoreType.DMA,
    ],
)
def _sc_kernel(*refs):
    _sc_body(*refs)


@jax.jit
def kernel(user_seq, spd_table, emb):
    seq = user_seq.astype(jnp.int32)
    seq_p = jnp.zeros((B, LP), jnp.int32).at[:, :L].set(seq)
    spd_flat = spd_table.reshape(-1)
    embt = emb.T.reshape(-1)
    out = _sc_kernel(seq_p, spd_flat, embt)
    return out.reshape(B, H, L, L)
